# baseline (speedup 1.0000x reference)
import functools

import jax
import jax.numpy as jnp
from jax import lax
from jax.experimental import pallas as pl
from jax.experimental.pallas import tpu as pltpu
from jax.experimental.pallas import tpu_sc as plsc

_E = 256
_C = 128
_S = 64
_NREP = 64
_NW = 32
_TAB_ROWS = _E * _NREP
_QROWS = 64
_NBUF = 4


def _sc_body(lat_hbm, sel_hbm, tab_hbm, out_hbm,
             sel_v, idx_v, lat_v, emb_v, sem_l, sem_g, sem_o):
    wid = lax.axis_index("s") * 2 + lax.axis_index("c")
    b_per_w = lat_hbm.shape[0] // _NW
    base = wid * b_per_w
    nsteps = b_per_w

    pltpu.sync_copy(sel_hbm.at[pl.ds(base, b_per_w)], sel_v)

    lane = lax.iota(jnp.int32, 16)
    rep4 = lax.shift_right_logical(lane, 2)
    gdn = lax.GatherDimensionNumbers(
        offset_dims=(), collapsed_slice_dims=(0,), start_index_map=(0,))

    def start(t, q):
        b = base + t
        s16 = sel_v[t, pl.ds(q * 16, 16)]
        for c in range(4):
            sval = lax.gather(s16, (4 * c + rep4)[:, None], gdn, (1,),
                              mode=lax.GatherScatterMode.PROMISE_IN_BOUNDS)
            i16 = (q * _QROWS + c * 16) + lane
            idx_v[q, pl.ds(c * 16, 16)] = i16 * _NREP + sval
        pltpu.async_copy(
            lat_hbm.at[b, pl.ds(q * _QROWS, _QROWS)], lat_v.at[q], sem_l.at[q])
        pltpu.async_copy(tab_hbm.at[idx_v.at[q]], emb_v.at[q], sem_g.at[q])

    def finish(t, q):
        b = base + t
        pltpu.make_async_copy(
            lat_hbm.at[b, pl.ds(q * _QROWS, _QROWS)], lat_v.at[q],
            sem_l.at[q]).wait()
        pltpu.make_async_copy(
            tab_hbm.at[idx_v.at[q]], emb_v.at[q], sem_g.at[q]).wait()

        def addrow(r, inner):
            for rr in range(2):
                for ch in range(8):
                    sl = pl.ds(ch * 16, 16)
                    emb_v[q, 2 * r + rr, sl] = (
                        emb_v[q, 2 * r + rr, sl] + lat_v[q, 2 * r + rr, sl])
            return inner

        lax.fori_loop(0, _QROWS // 2, addrow, 0)
        pltpu.async_copy(
            emb_v.at[q], out_hbm.at[b, pl.ds(q * _QROWS, _QROWS)], sem_o.at[q])

    def drain_out(t, q):
        pltpu.make_async_copy(
            emb_v.at[q], out_hbm.at[base + t, pl.ds(q * _QROWS, _QROWS)],
            sem_o.at[q]).wait()

    start(0, 0)
    start(0, 1)
    start(0, 2)

    def step(t, carry):
        for p in range(4):
            finish(t, p)
            if p == 0:
                @pl.when(t >= 1)
                def _():
                    drain_out(t - 1, 3)
            else:
                drain_out(t, p - 1)
            if p == 0:
                start(t, 3)
            else:
                @pl.when(t < nsteps - 1)
                def _():
                    start(t + 1, p - 1)
        return carry

    lax.fori_loop(0, nsteps, step, 0)
    drain_out(nsteps - 1, 3)


def kernel(latents, selections, emb):
    b = latents.shape[0]
    sel = selections.astype(jnp.int32)
    tab = emb.reshape(_TAB_ROWS, _C)
    b_per_w = b // _NW
    run = functools.partial(
        pl.kernel,
        mesh=plsc.VectorSubcoreMesh(core_axis_name="c", subcore_axis_name="s"),
        out_type=jax.ShapeDtypeStruct((b, _E, _C), jnp.float32),
        scratch_types=[
            pltpu.VMEM((b_per_w, _S), jnp.int32),
            pltpu.VMEM((_NBUF, _QROWS), jnp.int32),
            pltpu.VMEM((_NBUF, _QROWS, _C), jnp.float32),
            pltpu.VMEM((_NBUF, _QROWS, _C), jnp.float32),
            pltpu.SemaphoreType.DMA((_NBUF,)),
            pltpu.SemaphoreType.DMA((_NBUF,)),
            pltpu.SemaphoreType.DMA((_NBUF,)),
        ],
    )(_sc_body)
    return run(latents, sel, tab)
